# Initial kernel scaffold; baseline (speedup 1.0000x reference)
import functools

import jax
import jax.numpy as jnp
from jax import lax
from jax.experimental import pallas as pl
from jax.experimental.pallas import tpu as pltpu
from jax.experimental.pallas import tpu_sc as plsc

N = 10000
E = 320000
D = 128
H = 128
C = 64

NS = 16
EPT = E // NS
KE = 80
SUB = 25
NCH = EPT // KE
NBB = NCH // SUB
RPT = N // NS
CPR = 125
LANES = 16

_mesh = plsc.VectorSubcoreMesh(core_axis_name="c", subcore_axis_name="s")


@functools.partial(
    pl.kernel,
    out_type=jax.ShapeDtypeStruct((2, N, H), jnp.float32),
    mesh=_mesh,
    scratch_types=[
        pltpu.VMEM((SUB, KE), jnp.int32),
        pltpu.VMEM((SUB, KE), jnp.int32),
        pltpu.VMEM((SUB, KE), jnp.float32),
        pltpu.VMEM((KE, H), jnp.float32),
        pltpu.VMEM((CPR, H), jnp.float32),
        pltpu.VMEM_SHARED((N, H), jnp.float32),
        pltpu.SemaphoreType.DMA,
    ],
)
def _spmm_pair(s2_hbm, src_hbm, dst_hbm, w_hbm, out_hbm,
               src_v, dst_v, w_v, rows_v, cp_v, acc, sem):
    c = lax.axis_index("c")
    t = lax.axis_index("s")
    zero = jnp.zeros((LANES,), jnp.float32)

    def zrow(r, carry):
        for j in range(H // LANES):
            cp_v[r, pl.ds(j * LANES, LANES)] = zero
        return carry
    lax.fori_loop(0, CPR, zrow, 0)
    for kblk in range(RPT // CPR):
        pltpu.sync_copy(cp_v, acc.at[pl.ds(t * RPT + kblk * CPR, CPR)])
    plsc.subcore_barrier()

    def big(bb, carry):
        pltpu.sync_copy(src_hbm.at[c, t, pl.ds(bb * SUB, SUB)], src_v)
        pltpu.sync_copy(dst_hbm.at[c, t, pl.ds(bb * SUB, SUB)], dst_v)
        pltpu.sync_copy(w_hbm.at[c, t, pl.ds(bb * SUB, SUB)], w_v)

        def sub(i, carry2):
            pltpu.async_copy(s2_hbm.at[src_v.at[i]], rows_v, sem).wait()

            def mul(e, carry3):
                w = w_v[i, e]
                for j in range(H // LANES):
                    sl = pl.ds(j * LANES, LANES)
                    rows_v[e, sl] = rows_v[e, sl] * w
                return carry3
            lax.fori_loop(0, KE, mul, 0)
            pltpu.sync_copy(rows_v, acc.at[dst_v.at[i]], add=True)
            return carry2
        lax.fori_loop(0, SUB, sub, 0)
        return carry
    lax.fori_loop(0, NBB, big, 0)
    plsc.subcore_barrier()

    for kblk in range(RPT // CPR):
        r0 = t * RPT + kblk * CPR
        pltpu.sync_copy(acc.at[pl.ds(r0, CPR)], cp_v)
        pltpu.sync_copy(cp_v, out_hbm.at[c, pl.ds(r0, CPR)])


BM = 1000


def _mm_body(x_ref, w_ref, o_ref):
    o_ref[...] = jnp.dot(x_ref[0], w_ref[0],
                         preferred_element_type=jnp.float32)[None]


def _mm(xs, ws):
    return pl.pallas_call(
        _mm_body,
        grid=(2, N // BM),
        in_specs=[
            pl.BlockSpec((1, BM, D), lambda b, m: (b, m, 0)),
            pl.BlockSpec((1, D, H), lambda b, m: (b, 0, 0)),
        ],
        out_specs=pl.BlockSpec((1, BM, H), lambda b, m: (b, m, 0)),
        out_shape=jax.ShapeDtypeStruct((2, N, H), jnp.float32),
    )(xs, ws)


def _relu_mm_body(a_ref, b_ref, w_ref, o_ref):
    h = jnp.maximum(a_ref[0] + b_ref[0], 0.0)
    o_ref[...] = jnp.dot(h, w_ref[0], preferred_element_type=jnp.float32)[None]


def _relu_mm(aggs, biases, ws):
    return pl.pallas_call(
        _relu_mm_body,
        grid=(2, N // BM),
        in_specs=[
            pl.BlockSpec((1, BM, H), lambda b, m: (b, m, 0)),
            pl.BlockSpec((1, 1, H), lambda b, m: (b, 0, 0)),
            pl.BlockSpec((1, H, H), lambda b, m: (b, 0, 0)),
        ],
        out_specs=pl.BlockSpec((1, BM, H), lambda b, m: (b, m, 0)),
        out_shape=jax.ShapeDtypeStruct((2, N, H), jnp.float32),
    )(aggs, biases, ws)


def _log_softmax(t):
    mx = jnp.max(t, axis=1, keepdims=True)
    return t - mx - jnp.log(jnp.sum(jnp.exp(t - mx), axis=1, keepdims=True))


def _head_body(agg_ref, b2_ref, lw_ref, lb_ref, mw_ref, mb_ref, o_ref):
    lsm = []
    for b in range(2):
        t = agg_ref[b] + b2_ref[b]
        t = jnp.dot(t, lw_ref[b], preferred_element_type=jnp.float32) + lb_ref[b]
        lsm.append(_log_softmax(t))
    h = (jnp.dot(lsm[0], mw_ref[:H], preferred_element_type=jnp.float32)
         + jnp.dot(lsm[1], mw_ref[H:], preferred_element_type=jnp.float32)
         + mb_ref[...])
    o_ref[...] = _log_softmax(h)


def _head(agg2, b2s, lws, lbs, mw, mb):
    return pl.pallas_call(
        _head_body,
        grid=(N // BM,),
        in_specs=[
            pl.BlockSpec((2, BM, H), lambda m: (0, m, 0)),
            pl.BlockSpec((2, 1, H), lambda m: (0, 0, 0)),
            pl.BlockSpec((2, H, H), lambda m: (0, 0, 0)),
            pl.BlockSpec((2, 1, H), lambda m: (0, 0, 0)),
            pl.BlockSpec((2 * H, C), lambda m: (0, 0)),
            pl.BlockSpec((1, C), lambda m: (0, 0)),
        ],
        out_specs=pl.BlockSpec((BM, C), lambda m: (m, 0)),
        out_shape=jax.ShapeDtypeStruct((N, C), jnp.float32),
    )(agg2, b2s, lws, lbs, mw, mb)


def kernel(x0, x1, edge_index0, edge_weight0, edge_index1, edge_weight1,
           gcn_W1, gcn_b1, gcn_W2, gcn_b2, gcn_Lw, gcn_Lb,
           gin_W1, gin_b1, gin_W2, gin_b2, gin_Lw, gin_Lb,
           mix_Lw, mix_Lb):
    xs = jnp.stack([x0, x1])
    w1s = jnp.stack([gcn_W1, gin_W1])
    b1s = jnp.stack([gcn_b1, gin_b1]).reshape(2, 1, H)
    w2s = jnp.stack([gcn_W2, gin_W2])
    b2s = jnp.stack([gcn_b2, gin_b2]).reshape(2, 1, H)
    lws = jnp.stack([gcn_Lw, gin_Lw])
    lbs = jnp.stack([gcn_Lb, gin_Lb]).reshape(2, 1, H)
    mb = mix_Lb.reshape(1, C)

    src = jnp.stack([edge_index0[0], edge_index1[0] + N]).reshape(2, NS, NCH, KE)
    dst = jnp.stack([edge_index0[1], edge_index1[1]]).reshape(2, NS, NCH, KE)
    ws = jnp.stack([edge_weight0, edge_weight1]).reshape(2, NS, NCH, KE)

    sup1 = _mm(xs, w1s)
    agg1 = _spmm_pair(sup1.reshape(2 * N, H), src, dst, ws)
    sup2 = _relu_mm(agg1, b1s, w2s)
    agg2 = _spmm_pair(sup2.reshape(2 * N, H), src, dst, ws)
    return _head(agg2, b2s, lws, lbs, mix_Lw, mb)


# speedup vs baseline: 2.4018x; 2.4018x over previous
import functools

import jax
import jax.numpy as jnp
from jax import lax
from jax.experimental import pallas as pl
from jax.experimental.pallas import tpu as pltpu
from jax.experimental.pallas import tpu_sc as plsc

N = 10000
E = 320000
D = 128
H = 128
C = 64

NS = 16
EPT = E // NS
KE = 128
NCH = 160
EPTP = NCH * KE
PAD = EPTP - EPT
SUB = 16
NBB = NCH // SUB
CB = 80
NCB = N // CB
LANES = 16

def _spmm_body(s2_hbm, src_hbm, dst_hbm, w_hbm, out_hbm,
               src_v, dst_v, w_v, rows_v, cp_v, acc, sem):
    c = lax.axis_index("c")
    t = lax.axis_index("s")
    zero = jnp.zeros((LANES,), jnp.float32)

    def zrow(r, carry):
        for j in range(H // LANES):
            cp_v[r, pl.ds(j * LANES, LANES)] = zero
        return carry
    lax.fori_loop(0, CB, zrow, 0)
    nblk = (NCB - t + NS - 1) // NS

    def zblk(kk, carry):
        pltpu.sync_copy(cp_v, acc.at[pl.ds((t + kk * NS) * CB, CB)])
        return carry
    lax.fori_loop(0, nblk, zblk, 0)
    plsc.subcore_barrier()

    def big(bb, carry):
        pltpu.sync_copy(src_hbm.at[c, t, pl.ds(bb * SUB, SUB)], src_v)
        pltpu.sync_copy(dst_hbm.at[c, t, pl.ds(bb * SUB, SUB)], dst_v)
        pltpu.sync_copy(w_hbm.at[c, t, pl.ds(bb * SUB, SUB)], w_v)

        def sub(i, carry2):
            pltpu.async_copy(s2_hbm.at[src_v.at[i]], rows_v, sem).wait()

            def mul(g, carry3):
                wvec = w_v[i, pl.ds(g * LANES, LANES)]
                for l in range(LANES):
                    w = wvec[l]
                    e = g * LANES + l
                    for j in range(H // LANES):
                        sl = pl.ds(j * LANES, LANES)
                        rows_v[e, sl] = rows_v[e, sl] * w
                return carry3
            lax.fori_loop(0, KE // LANES, mul, 0)
            pltpu.sync_copy(rows_v, acc.at[dst_v.at[i]], add=True)
            return carry2
        lax.fori_loop(0, SUB, sub, 0)
        return carry
    lax.fori_loop(0, NBB, big, 0)
    plsc.subcore_barrier()

    def cblk(kk, carry):
        r0 = (t + kk * NS) * CB
        pltpu.sync_copy(acc.at[pl.ds(r0, CB)], cp_v)
        pltpu.sync_copy(cp_v, out_hbm.at[c, pl.ds(r0, CB)])
        return carry
    lax.fori_loop(0, nblk, cblk, 0)


@functools.cache
def _spmm_pair_fn():
    mesh = plsc.VectorSubcoreMesh(core_axis_name="c", subcore_axis_name="s")
    return pl.kernel(
        _spmm_body,
        out_type=jax.ShapeDtypeStruct((2, N, H), jnp.float32),
        mesh=mesh,
        scratch_types=[
            pltpu.VMEM((SUB, KE), jnp.int32),
            pltpu.VMEM((SUB, KE), jnp.int32),
            pltpu.VMEM((SUB, KE), jnp.float32),
            pltpu.VMEM((KE, H), jnp.float32),
            pltpu.VMEM((CB, H), jnp.float32),
            pltpu.VMEM_SHARED((N, H), jnp.float32),
            pltpu.SemaphoreType.DMA,
        ],
    )


def _spmm_pair(s2, src, dst, w):
    return _spmm_pair_fn()(s2, src, dst, w)


BM = 1000


def _mm_body(x_ref, w_ref, o_ref):
    o_ref[...] = jnp.dot(x_ref[0], w_ref[0],
                         preferred_element_type=jnp.float32)[None]


def _mm(xs, ws):
    return pl.pallas_call(
        _mm_body,
        grid=(2, N // BM),
        in_specs=[
            pl.BlockSpec((1, BM, D), lambda b, m: (b, m, 0)),
            pl.BlockSpec((1, D, H), lambda b, m: (b, 0, 0)),
        ],
        out_specs=pl.BlockSpec((1, BM, H), lambda b, m: (b, m, 0)),
        out_shape=jax.ShapeDtypeStruct((2, N, H), jnp.float32),
    )(xs, ws)


def _relu_mm_body(a_ref, b_ref, w_ref, o_ref):
    h = jnp.maximum(a_ref[0] + b_ref[0], 0.0)
    o_ref[...] = jnp.dot(h, w_ref[0], preferred_element_type=jnp.float32)[None]


def _relu_mm(aggs, biases, ws):
    return pl.pallas_call(
        _relu_mm_body,
        grid=(2, N // BM),
        in_specs=[
            pl.BlockSpec((1, BM, H), lambda b, m: (b, m, 0)),
            pl.BlockSpec((1, 1, H), lambda b, m: (b, 0, 0)),
            pl.BlockSpec((1, H, H), lambda b, m: (b, 0, 0)),
        ],
        out_specs=pl.BlockSpec((1, BM, H), lambda b, m: (b, m, 0)),
        out_shape=jax.ShapeDtypeStruct((2, N, H), jnp.float32),
    )(aggs, biases, ws)


def _log_softmax(t):
    mx = jnp.max(t, axis=1, keepdims=True)
    return t - mx - jnp.log(jnp.sum(jnp.exp(t - mx), axis=1, keepdims=True))


def _head_body(agg_ref, b2_ref, lw_ref, lb_ref, mw_ref, mb_ref, o_ref):
    lsm = []
    for b in range(2):
        t = agg_ref[b] + b2_ref[b]
        t = jnp.dot(t, lw_ref[b], preferred_element_type=jnp.float32) + lb_ref[b]
        lsm.append(_log_softmax(t))
    h = (jnp.dot(lsm[0], mw_ref[:H], preferred_element_type=jnp.float32)
         + jnp.dot(lsm[1], mw_ref[H:], preferred_element_type=jnp.float32)
         + mb_ref[...])
    o_ref[...] = _log_softmax(h)


def _head(agg2, b2s, lws, lbs, mw, mb):
    return pl.pallas_call(
        _head_body,
        grid=(N // BM,),
        in_specs=[
            pl.BlockSpec((2, BM, H), lambda m: (0, m, 0)),
            pl.BlockSpec((2, 1, H), lambda m: (0, 0, 0)),
            pl.BlockSpec((2, H, H), lambda m: (0, 0, 0)),
            pl.BlockSpec((2, 1, H), lambda m: (0, 0, 0)),
            pl.BlockSpec((2 * H, C), lambda m: (0, 0)),
            pl.BlockSpec((1, C), lambda m: (0, 0)),
        ],
        out_specs=pl.BlockSpec((BM, C), lambda m: (m, 0)),
        out_shape=jax.ShapeDtypeStruct((N, C), jnp.float32),
    )(agg2, b2s, lws, lbs, mw, mb)


def kernel(x0, x1, edge_index0, edge_weight0, edge_index1, edge_weight1,
           gcn_W1, gcn_b1, gcn_W2, gcn_b2, gcn_Lw, gcn_Lb,
           gin_W1, gin_b1, gin_W2, gin_b2, gin_Lw, gin_Lb,
           mix_Lw, mix_Lb):
    xs = jnp.stack([x0, x1])
    w1s = jnp.stack([gcn_W1, gin_W1])
    b1s = jnp.stack([gcn_b1, gin_b1]).reshape(2, 1, H)
    w2s = jnp.stack([gcn_W2, gin_W2])
    b2s = jnp.stack([gcn_b2, gin_b2]).reshape(2, 1, H)
    lws = jnp.stack([gcn_Lw, gin_Lw])
    lbs = jnp.stack([gcn_Lb, gin_Lb]).reshape(2, 1, H)
    mb = mix_Lb.reshape(1, C)

    def lay(a):
        a = a.reshape(2, NS, EPT)
        return jnp.pad(a, ((0, 0), (0, 0), (0, PAD))).reshape(2, NS, NCH, KE)

    src = lay(jnp.stack([edge_index0[0], edge_index1[0] + N]))
    dst = lay(jnp.stack([edge_index0[1], edge_index1[1]]))
    ws = lay(jnp.stack([edge_weight0, edge_weight1]))

    sup1 = _mm(xs, w1s)
    agg1 = _spmm_pair(sup1.reshape(2 * N, H), src, dst, ws)
    sup2 = _relu_mm(agg1, b1s, w2s)
    agg2 = _spmm_pair(sup2.reshape(2 * N, H), src, dst, ws)
    return _head(agg2, b2s, lws, lbs, mix_Lw, mb)
